# baseline (speedup 1.0000x reference)
import functools

import jax
import jax.numpy as jnp
from jax import lax
from jax.experimental import pallas as pl
from jax.experimental.pallas import tpu as pltpu
from jax.experimental.pallas import tpu_sc as plsc

MARGIN_ = 1.0
NC, NS, L = 2, 16, 16
NW = NC * NS
PAIRS = 16384
PW = PAIRS // NW
STEP = 128
NSTEPS = PW // STEP
JROWS = 8
D = 64
DP = 128
NE = 1000000
NBUK = (NE + DP - 1) // DP
BPW = 245
OUTR = 500096


def _lane_perm(x, idx):
    dnums = lax.GatherDimensionNumbers(
        offset_dims=(), collapsed_slice_dims=(0,), start_index_map=(0,))
    return lax.gather(x, idx[:, None], dnums, (1,),
                      mode=lax.GatherScatterMode.PROMISE_IN_BOUNDS)


NDEEP = 4


def _transpose_kernel(entT, relT, out1, out2, *bufs):
    ins = bufs[0:NDEEP]
    touts = bufs[NDEEP:2 * NDEEP]
    sis = bufs[2 * NDEEP:3 * NDEEP]
    sos = bufs[3 * NDEEP:4 * NDEEP]
    wid = lax.axis_index("s") * NC + lax.axis_index("c")
    lo = wid * BPW
    lanes = lax.iota(jnp.int32, L)

    def stream(tab, out):
        def bucket(i):
            return jnp.minimum(lo + i, NBUK - 1)

        def fire_in(bi, buf, sem):
            col = pl.multiple_of(bucket(bi) * DP, DP)
            return pltpu.async_copy(tab.at[:, pl.ds(col, DP)], buf, sem)

        def fire_out(bi, buf, sem):
            row = pl.multiple_of(bucket(bi) * D, D)
            return pltpu.async_copy(buf, out.at[pl.ds(row, D)], sem)

        def wait_like_in(buf, sem):
            pltpu.make_async_copy(tab.at[:, pl.ds(0, DP)], buf, sem).wait()

        def wait_like_out(buf, sem):
            pltpu.make_async_copy(buf, out.at[pl.ds(0, D)], sem).wait()

        cjs = [(lanes + j) & (L - 1) for j in range(L)]
        wjs = [cjs[j] * D + lanes for j in range(L)]

        def transpose(src, dst):
            def subtile(s, _):
                r0 = (s & 3) * L
                c0 = (s >> 2) * L
                rows = r0 + lanes
                base = c0 * D + r0
                for j in range(L):
                    v = plsc.load_gather(src, [rows, cjs[j] + c0])
                    flat = wjs[j] + base
                    plsc.store_scatter(
                        dst, [lax.shift_right_logical(flat, 7),
                              lax.bitwise_and(flat, DP - 1)], v)
                return 0

            lax.fori_loop(0, (D // L) * (DP // L), subtile, 0)

        for u in range(NDEEP):
            fire_in(u, ins[u], sis[u])
            pltpu.async_copy(touts[u], out.at[pl.ds(OUTR - D, D)], sos[u])

        def body(c, _):
            for u in range(NDEEP):
                b = NDEEP * c + u
                wait_like_in(ins[u], sis[u])
                wait_like_out(touts[u], sos[u])
                transpose(ins[u], touts[u])
                fire_out(b, touts[u], sos[u])
                fire_in(b + NDEEP, ins[u], sis[u])
            return 0

        lax.fori_loop(0, (BPW + NDEEP - 1) // NDEEP, body, 0)
        for u in range(NDEEP):
            wait_like_in(ins[u], sis[u])
            wait_like_out(touts[u], sos[u])

    stream(entT, out1)
    stream(relT, out2)


def _sc_loss_kernel(ent_hbm, rel_hbm, ps_h, pr_h, pt_h, ns_h, nr_h, nt_h,
                    out_hbm,
                    ps_v, pr_v, pt_v, ns_v, nr_v, nt_v,
                    fs_v, fr_v, ft_v, gs_v, gr_v, gt_v,
                    r_ps, r_pr, r_pt, r_ns, r_nr, r_nt,
                    acc_v, sem):
    wid = lax.axis_index("s") * NC + lax.axis_index("c")

    pltpu.sync_copy(ps_h.at[wid], ps_v)
    pltpu.sync_copy(pr_h.at[wid], pr_v)
    pltpu.sync_copy(pt_h.at[wid], pt_v)
    pltpu.sync_copy(ns_h.at[wid], ns_v)
    pltpu.sync_copy(nr_h.at[wid], nr_v)
    pltpu.sync_copy(nt_h.at[wid], nt_v)

    for ov, pv in ((ps_v, fs_v), (pr_v, fr_v), (pt_v, ft_v),
                   (ns_v, gs_v), (nr_v, gr_v), (nt_v, gt_v)):
        def shift_row(j, _, ov=ov, pv=pv):
            for c in range(STEP // L):
                sl = pl.ds(c * L, L)
                pv[j, sl] = lax.shift_right_logical(ov[j, sl], 1)
            return 0
        lax.fori_loop(0, NSTEPS, shift_row, 0)

    lanes = lax.iota(jnp.int32, L)
    acc = jnp.zeros((L,), jnp.float32)
    for j in range(NSTEPS):
        cps = pltpu.async_copy(ent_hbm.at[fs_v.at[j]], r_ps, sem)
        cpr = pltpu.async_copy(rel_hbm.at[fr_v.at[j]], r_pr, sem)
        cpt = pltpu.async_copy(ent_hbm.at[ft_v.at[j]], r_pt, sem)
        cns = pltpu.async_copy(ent_hbm.at[gs_v.at[j]], r_ns, sem)
        cnr = pltpu.async_copy(rel_hbm.at[gr_v.at[j]], r_nr, sem)
        cnt = pltpu.async_copy(ent_hbm.at[gt_v.at[j]], r_nt, sem)
        for c in (cps, cpr, cpt, cns, cnr, cnt):
            c.wait()

        def pair_block(pb, acc, j=j):
            chunks = [ov[j, pl.ds(pb * L, L)]
                      for ov in (ps_v, pr_v, pt_v, ns_v, nr_v, nt_v)]
            for u in range(L):
                p = pb * L + u
                offs = [lax.bitwise_and(ch[u], 1) * D for ch in chunks]
                d = jnp.zeros((L,), jnp.float32)
                for k in range(D // L):
                    b = k * L
                    xp = jnp.abs(r_ps[p, pl.ds(offs[0] + b, L)]
                                 + r_pr[p, pl.ds(offs[1] + b, L)]
                                 - r_pt[p, pl.ds(offs[2] + b, L)])
                    xn = jnp.abs(r_ns[p, pl.ds(offs[3] + b, L)]
                                 + r_nr[p, pl.ds(offs[4] + b, L)]
                                 - r_nt[p, pl.ds(offs[5] + b, L)])
                    d = d + (xp - xn)
                for sh in (8, 4, 2, 1):
                    d = d + _lane_perm(d, lanes ^ sh)
                acc = acc + jnp.maximum(d + MARGIN_, 0.0)
            return acc

        acc = lax.fori_loop(0, STEP // L, pair_block, acc)

    acc_v[pl.ds(0, L)] = jnp.where(lanes == 0, acc, 0.0)
    for z in range(1, DP // L):
        acc_v[pl.ds(z * L, L)] = jnp.zeros((L,), jnp.float32)
    pltpu.sync_copy(acc_v, out_hbm.at[wid])


@jax.jit
def kernel(train_indices, ent_embeds, rel_embeds):
    idx = train_indices.astype(jnp.int32)
    pos = idx[:PAIRS]
    neg = idx[PAIRS:]
    blocks = []
    for c in (pos[:, 0], pos[:, 1], pos[:, 2],
              neg[:, 0], neg[:, 1], neg[:, 2]):
        b = c.reshape(NW, NSTEPS, STEP)
        b = jnp.pad(b, ((0, 0), (0, JROWS - NSTEPS), (0, 0)))
        blocks.append(b)

    mesh = plsc.VectorSubcoreMesh(core_axis_name="c", subcore_axis_name="s")
    tke = functools.partial(
        pl.kernel,
        mesh=mesh,
        compiler_params=pltpu.CompilerParams(needs_layout_passes=False),
        out_type=(jax.ShapeDtypeStruct((OUTR, DP), jnp.float32),
                  jax.ShapeDtypeStruct((OUTR, DP), jnp.float32)),
        scratch_types=(
            [pltpu.VMEM((D, DP), jnp.float32)] * 8
            + [pltpu.SemaphoreType.DMA] * 8
        ),
    )(_transpose_kernel)
    ent2, rel2 = tke(ent_embeds.T, rel_embeds.T)

    run = functools.partial(
        pl.kernel,
        mesh=mesh,
        out_type=jax.ShapeDtypeStruct((NW, DP), jnp.float32),
        scratch_types=(
            [pltpu.VMEM((JROWS, STEP), jnp.int32)] * 12
            + [pltpu.VMEM((STEP, DP), jnp.float32)] * 6
            + [pltpu.VMEM((DP,), jnp.float32), pltpu.SemaphoreType.DMA]
        ),
    )(_sc_loss_kernel)
    partials = run(ent2, rel2, *blocks)
    return jnp.sum(partials) / PAIRS


# speedup vs baseline: 2.3387x; 2.3387x over previous
import functools

import jax
import jax.numpy as jnp
from jax import lax
from jax.experimental import pallas as pl
from jax.experimental.pallas import tpu as pltpu
from jax.experimental.pallas import tpu_sc as plsc

MARGIN_ = 1.0
NC, NS, L = 2, 16, 16
NW = NC * NS
PAIRS = 16384
PW = PAIRS // NW
STEP = 128
NSTEPS = PW // STEP
JROWS = 8
D = 64
DP = 128
NE = 1000000
NBUK = (NE + DP - 1) // DP
BPW = 245
OUTR = 500096


def _lane_perm(x, idx):
    dnums = lax.GatherDimensionNumbers(
        offset_dims=(), collapsed_slice_dims=(0,), start_index_map=(0,))
    return lax.gather(x, idx[:, None], dnums, (1,),
                      mode=lax.GatherScatterMode.PROMISE_IN_BOUNDS)


NDEEP = 4


def _transpose_kernel(entT, relT, out1, out2, *bufs):
    ins = bufs[0:NDEEP]
    touts = bufs[NDEEP:2 * NDEEP]
    sis = bufs[2 * NDEEP:3 * NDEEP]
    sos = bufs[3 * NDEEP:4 * NDEEP]
    wid = lax.axis_index("s") * NC + lax.axis_index("c")
    lo = wid * BPW
    lanes = lax.iota(jnp.int32, L)

    def stream(tab, out):
        def bucket(i):
            return jnp.minimum(lo + i, NBUK - 1)

        def fire_in(bi, buf, sem):
            col = pl.multiple_of(bucket(bi) * DP, DP)
            return pltpu.async_copy(tab.at[:, pl.ds(col, DP)], buf, sem)

        def fire_out(bi, buf, sem):
            row = pl.multiple_of(bucket(bi) * D, D)
            return pltpu.async_copy(buf, out.at[pl.ds(row, D)], sem)

        def wait_like_in(buf, sem):
            pltpu.make_async_copy(tab.at[:, pl.ds(0, DP)], buf, sem).wait()

        def wait_like_out(buf, sem):
            pltpu.make_async_copy(buf, out.at[pl.ds(0, D)], sem).wait()

        cjs = [(lanes + j) & (L - 1) for j in range(L)]
        rowp = [lax.shift_right_logical(cjs[j], 1) for j in range(L)]
        colp = [(cjs[j] & 1) * D + lanes for j in range(L)]

        def transpose(src, dst):
            def subtile(s, _):
                for h in range(2):
                    sh = s * 2 + h
                    r0 = (sh & 3) * L
                    c0 = (sh >> 2) * L
                    c0h = (sh >> 2) * (L // 2)
                    rows = r0 + lanes
                    vs = [plsc.load_gather(src, [rows, cjs[j] + c0])
                          for j in range(L)]
                    for j in range(L):
                        plsc.store_scatter(
                            dst, [rowp[j] + c0h, colp[j] + r0], vs[j])
                return 0

            lax.fori_loop(0, (D // L) * (DP // L) // 2, subtile, 0)

        for u in range(NDEEP):
            fire_in(u, ins[u], sis[u])
            pltpu.async_copy(touts[u], out.at[pl.ds(OUTR - D, D)], sos[u])

        def body(c, _):
            for u in range(NDEEP):
                b = NDEEP * c + u
                wait_like_in(ins[u], sis[u])
                wait_like_out(touts[u], sos[u])
                transpose(ins[u], touts[u])
                fire_out(b, touts[u], sos[u])
                fire_in(b + NDEEP, ins[u], sis[u])
            return 0

        lax.fori_loop(0, (BPW + NDEEP - 1) // NDEEP, body, 0)
        for u in range(NDEEP):
            wait_like_in(ins[u], sis[u])
            wait_like_out(touts[u], sos[u])

    stream(entT, out1)
    stream(relT, out2)


def _sc_loss_kernel(ent_hbm, rel_hbm, ps_h, pr_h, pt_h, ns_h, nr_h, nt_h,
                    out_hbm,
                    ps_v, pr_v, pt_v, ns_v, nr_v, nt_v,
                    fs_v, fr_v, ft_v, gs_v, gr_v, gt_v,
                    r_ps, r_pr, r_pt, r_ns, r_nr, r_nt,
                    acc_v, sem):
    wid = lax.axis_index("s") * NC + lax.axis_index("c")

    pltpu.sync_copy(ps_h.at[wid], ps_v)
    pltpu.sync_copy(pr_h.at[wid], pr_v)
    pltpu.sync_copy(pt_h.at[wid], pt_v)
    pltpu.sync_copy(ns_h.at[wid], ns_v)
    pltpu.sync_copy(nr_h.at[wid], nr_v)
    pltpu.sync_copy(nt_h.at[wid], nt_v)

    for ov, pv in ((ps_v, fs_v), (pr_v, fr_v), (pt_v, ft_v),
                   (ns_v, gs_v), (nr_v, gr_v), (nt_v, gt_v)):
        def shift_row(j, _, ov=ov, pv=pv):
            for c in range(STEP // L):
                sl = pl.ds(c * L, L)
                pv[j, sl] = lax.shift_right_logical(ov[j, sl], 1)
            return 0
        lax.fori_loop(0, NSTEPS, shift_row, 0)

    lanes = lax.iota(jnp.int32, L)
    acc = jnp.zeros((L,), jnp.float32)
    for j in range(NSTEPS):
        cps = pltpu.async_copy(ent_hbm.at[fs_v.at[j]], r_ps, sem)
        cpr = pltpu.async_copy(rel_hbm.at[fr_v.at[j]], r_pr, sem)
        cpt = pltpu.async_copy(ent_hbm.at[ft_v.at[j]], r_pt, sem)
        cns = pltpu.async_copy(ent_hbm.at[gs_v.at[j]], r_ns, sem)
        cnr = pltpu.async_copy(rel_hbm.at[gr_v.at[j]], r_nr, sem)
        cnt = pltpu.async_copy(ent_hbm.at[gt_v.at[j]], r_nt, sem)
        for c in (cps, cpr, cpt, cns, cnr, cnt):
            c.wait()

        def pair_block(pb, acc, j=j):
            chunks = [ov[j, pl.ds(pb * L, L)]
                      for ov in (ps_v, pr_v, pt_v, ns_v, nr_v, nt_v)]
            for u in range(L):
                p = pb * L + u
                offs = [lax.bitwise_and(ch[u], 1) * D for ch in chunks]
                d = jnp.zeros((L,), jnp.float32)
                for k in range(D // L):
                    b = k * L
                    xp = jnp.abs(r_ps[p, pl.ds(offs[0] + b, L)]
                                 + r_pr[p, pl.ds(offs[1] + b, L)]
                                 - r_pt[p, pl.ds(offs[2] + b, L)])
                    xn = jnp.abs(r_ns[p, pl.ds(offs[3] + b, L)]
                                 + r_nr[p, pl.ds(offs[4] + b, L)]
                                 - r_nt[p, pl.ds(offs[5] + b, L)])
                    d = d + (xp - xn)
                for sh in (8, 4, 2, 1):
                    d = d + _lane_perm(d, lanes ^ sh)
                acc = acc + jnp.maximum(d + MARGIN_, 0.0)
            return acc

        acc = lax.fori_loop(0, STEP // L, pair_block, acc)

    acc_v[pl.ds(0, L)] = jnp.where(lanes == 0, acc, 0.0)
    for z in range(1, DP // L):
        acc_v[pl.ds(z * L, L)] = jnp.zeros((L,), jnp.float32)
    pltpu.sync_copy(acc_v, out_hbm.at[wid])


@jax.jit
def kernel(train_indices, ent_embeds, rel_embeds):
    idx = train_indices.astype(jnp.int32)
    pos = idx[:PAIRS]
    neg = idx[PAIRS:]
    blocks = []
    for c in (pos[:, 0], pos[:, 1], pos[:, 2],
              neg[:, 0], neg[:, 1], neg[:, 2]):
        b = c.reshape(NW, NSTEPS, STEP)
        b = jnp.pad(b, ((0, 0), (0, JROWS - NSTEPS), (0, 0)))
        blocks.append(b)

    mesh = plsc.VectorSubcoreMesh(core_axis_name="c", subcore_axis_name="s")
    tke = functools.partial(
        pl.kernel,
        mesh=mesh,
        compiler_params=pltpu.CompilerParams(needs_layout_passes=False),
        out_type=(jax.ShapeDtypeStruct((OUTR, DP), jnp.float32),
                  jax.ShapeDtypeStruct((OUTR, DP), jnp.float32)),
        scratch_types=(
            [pltpu.VMEM((D, DP), jnp.float32)] * 8
            + [pltpu.SemaphoreType.DMA] * 8
        ),
    )(_transpose_kernel)
    ent2, rel2 = tke(ent_embeds.T, rel_embeds.T)

    run = functools.partial(
        pl.kernel,
        mesh=mesh,
        out_type=jax.ShapeDtypeStruct((NW, DP), jnp.float32),
        scratch_types=(
            [pltpu.VMEM((JROWS, STEP), jnp.int32)] * 12
            + [pltpu.VMEM((STEP, DP), jnp.float32)] * 6
            + [pltpu.VMEM((DP,), jnp.float32), pltpu.SemaphoreType.DMA]
        ),
    )(_sc_loss_kernel)
    partials = run(ent2, rel2, *blocks)
    return jnp.sum(partials) / PAIRS
